# baseline (speedup 1.0000x reference)
import functools

import jax
import jax.numpy as jnp
from jax import lax
from jax.experimental import pallas as pl
from jax.experimental.pallas import tpu as pltpu
from jax.experimental.pallas import tpu_sc as plsc

N = 10000
NP = 10240
E = 320000
D_IN = 128
D_HID = 128
D_OUT = 40
WH = 72
DEGC = 56

NC = 2
NS = 16
CHUNK = 512
CPS = 40
EPAD = NS * CPS * CHUNK

BM = 512
BM3 = 2000


def _make_edge_scatter(widths, k0):
    mesh = plsc.VectorSubcoreMesh(core_axis_name="c", subcore_axis_name="s")
    rps = NP // NS
    k1 = CPS - k0
    d = widths[0]
    assert all(w == d for w in widths)
    ntab = len(widths)

    @functools.partial(
        pl.kernel,
        mesh=mesh,
        compiler_params=pltpu.CompilerParams(use_tc_tiling_on_sc=False),
        out_type=tuple(jax.ShapeDtypeStruct((NC, NP, d), jnp.float32)
                       for _ in range(ntab)),
        scratch_types=[
            pltpu.VMEM((k0, CHUNK), jnp.int32),
            pltpu.VMEM((k0, CHUNK), jnp.int32),
            pltpu.VMEM((CHUNK, d), jnp.float32),
            pltpu.VMEM_SHARED((NP, d), jnp.float32),
            pltpu.SemaphoreType.DMA,
        ],
    )
    def edge_scatter(*refs):
        tabs = refs[:ntab]
        src_hbm, dst_hbm, zeros_hbm = refs[ntab:ntab + 3]
        outs = refs[ntab + 3:2 * ntab + 3]
        src_v, dst_v, rows_v, acc, gsem = refs[2 * ntab + 3:]
        cid = lax.axis_index("c")
        sid = lax.axis_index("s")

        @pl.when(cid == 0)
        def _():
            pltpu.sync_copy(src_hbm.at[sid, pl.ds(0, k0)], src_v)
            pltpu.sync_copy(dst_hbm.at[sid, pl.ds(0, k0)], dst_v)

        @pl.when(cid != 0)
        def _():
            pltpu.sync_copy(src_hbm.at[sid, pl.ds(k0, k1)],
                            src_v.at[pl.ds(0, k1)])
            pltpu.sync_copy(dst_hbm.at[sid, pl.ds(k0, k1)],
                            dst_v.at[pl.ds(0, k1)])

        nch = jnp.where(cid == 0, k0, k1)

        for tab_hbm, out_hbm in zip(tabs, outs):
            pltpu.sync_copy(zeros_hbm.at[pl.ds(sid * rps, rps)],
                            acc.at[pl.ds(sid * rps, rps)])
            plsc.subcore_barrier()

            def body(j, carry):
                pltpu.async_copy(tab_hbm.at[src_v.at[j]], rows_v, gsem).wait()
                pltpu.sync_copy(rows_v, acc.at[dst_v.at[j]], add=True)
                return carry

            lax.fori_loop(0, nch, body, 0)
            plsc.subcore_barrier()
            pltpu.sync_copy(acc.at[pl.ds(sid * rps, rps)],
                            out_hbm.at[cid, pl.ds(sid * rps, rps)])

    return edge_scatter


_scatter1 = _make_edge_scatter((WH, WH), 25)
_scatter2 = _make_edge_scatter((D_OUT,), 23)


def _mm1_body(x_ref, wl_ref, wr_ref, ta_ref, tb_ref, yr_ref):
    i = pl.program_id(0)
    xb = x_ref[...]
    mm = jnp.dot(xb, wl_ref[...], preferred_element_type=jnp.float32)
    rows = i * BM + lax.broadcasted_iota(jnp.int32, (BM, 8), 0)
    cols = lax.broadcasted_iota(jnp.int32, (BM, 8), 1)
    ones = jnp.where((rows < N) & (cols == 0), 1.0, 0.0)
    zeros8 = jnp.zeros((BM, 8), jnp.float32)
    ta_ref[...] = mm[:, :WH]
    tb_ref[...] = jnp.concatenate([mm[:, WH:], ones, zeros8], axis=1)
    yr_ref[...] = jnp.dot(xb, wr_ref[...], preferred_element_type=jnp.float32)


def _mid_body(aa_ref, ab_ref, yr_ref, b1_ref, w2l_ref, w2r_ref, b2_ref,
              y2_ref, r2_ref, dinv_ref):
    i = pl.program_id(0)
    sa = aa_ref[0] + aa_ref[1]
    sb = ab_ref[0] + ab_ref[1]
    deg = sb[:, DEGC:DEGC + 1]
    dinv = 1.0 / jnp.maximum(deg, 1.0)
    agg = jnp.concatenate([sa, sb[:, :DEGC]], axis=1)
    h = agg * dinv + b1_ref[...] + yr_ref[...]
    h = jnp.maximum(h, 0.0)
    rows = i * BM + lax.broadcasted_iota(jnp.int32, (BM, D_HID), 0)
    h = jnp.where(rows < N, h, 0.0)
    y2_ref[...] = jnp.dot(h, w2l_ref[...], preferred_element_type=jnp.float32)
    r2_ref[...] = (jnp.dot(h, w2r_ref[...], preferred_element_type=jnp.float32)
                   + b2_ref[...])
    dinv_ref[...] = jnp.broadcast_to(dinv, (BM, 8))


def _out_body(a_ref, dinv_ref, r2_ref, o_ref):
    s = a_ref[0] + a_ref[1]
    o_ref[...] = s * dinv_ref[...][:, :1] + r2_ref[...]


def kernel(x, edge_index, W1_l, b1_l, W1_r, W2_l, b2_l, W2_r):
    src = edge_index[0].astype(jnp.int32)
    dst = edge_index[1].astype(jnp.int32)
    src_p = (jnp.full((EPAD,), N, jnp.int32).at[:E].set(src)
             .reshape(NS, CPS, CHUNK))
    dst_p = (jnp.zeros((EPAD,), jnp.int32).at[:E].set(dst)
             .reshape(NS, CPS, CHUNK))
    x_p = jnp.zeros((NP, D_IN), jnp.float32).at[:N].set(x)
    zeros1 = jnp.zeros((NP, WH), jnp.float32)
    zeros2 = jnp.zeros((NP, D_OUT), jnp.float32)
    b1_2d = b1_l.reshape(1, D_HID)
    b2_2d = b2_l.reshape(1, D_OUT)

    ytabA, ytabB, yr1 = pl.pallas_call(
        _mm1_body,
        grid=(NP // BM,),
        in_specs=[
            pl.BlockSpec((BM, D_IN), lambda i: (i, 0)),
            pl.BlockSpec((D_IN, D_HID), lambda i: (0, 0)),
            pl.BlockSpec((D_IN, D_HID), lambda i: (0, 0)),
        ],
        out_specs=[
            pl.BlockSpec((BM, WH), lambda i: (i, 0)),
            pl.BlockSpec((BM, WH), lambda i: (i, 0)),
            pl.BlockSpec((BM, D_HID), lambda i: (i, 0)),
        ],
        out_shape=[
            jax.ShapeDtypeStruct((NP, WH), jnp.float32),
            jax.ShapeDtypeStruct((NP, WH), jnp.float32),
            jax.ShapeDtypeStruct((NP, D_HID), jnp.float32),
        ],
    )(x_p, W1_l, W1_r)

    accA, accB = _scatter1(ytabA, ytabB, src_p, dst_p, zeros1)

    y2tab, r2, dinv = pl.pallas_call(
        _mid_body,
        grid=(NP // BM,),
        in_specs=[
            pl.BlockSpec((NC, BM, WH), lambda i: (0, i, 0)),
            pl.BlockSpec((NC, BM, WH), lambda i: (0, i, 0)),
            pl.BlockSpec((BM, D_HID), lambda i: (i, 0)),
            pl.BlockSpec((1, D_HID), lambda i: (0, 0)),
            pl.BlockSpec((D_HID, D_OUT), lambda i: (0, 0)),
            pl.BlockSpec((D_HID, D_OUT), lambda i: (0, 0)),
            pl.BlockSpec((1, D_OUT), lambda i: (0, 0)),
        ],
        out_specs=[
            pl.BlockSpec((BM, D_OUT), lambda i: (i, 0)),
            pl.BlockSpec((BM, D_OUT), lambda i: (i, 0)),
            pl.BlockSpec((BM, 8), lambda i: (i, 0)),
        ],
        out_shape=[
            jax.ShapeDtypeStruct((NP, D_OUT), jnp.float32),
            jax.ShapeDtypeStruct((NP, D_OUT), jnp.float32),
            jax.ShapeDtypeStruct((NP, 8), jnp.float32),
        ],
    )(accA, accB, yr1, b1_2d, W2_l, W2_r, b2_2d)

    (acc2,) = _scatter2(y2tab, src_p, dst_p, zeros2)

    out = pl.pallas_call(
        _out_body,
        grid=(N // BM3,),
        in_specs=[
            pl.BlockSpec((NC, BM3, D_OUT), lambda i: (0, i, 0)),
            pl.BlockSpec((BM3, 8), lambda i: (i, 0)),
            pl.BlockSpec((BM3, D_OUT), lambda i: (i, 0)),
        ],
        out_specs=pl.BlockSpec((BM3, D_OUT), lambda i: (i, 0)),
        out_shape=jax.ShapeDtypeStruct((N, D_OUT), jnp.float32),
    )(acc2, dinv, r2)

    return (out, edge_index)


# speedup vs baseline: 1.6599x; 1.6599x over previous
import functools

import jax
import jax.numpy as jnp
from jax import lax
from jax.experimental import pallas as pl
from jax.experimental.pallas import tpu as pltpu
from jax.experimental.pallas import tpu_sc as plsc

N = 10000
NP = 10240
E = 320000
D_IN = 128
D_HID = 128
D_OUT = 40
W1TAB = 136

NC = 2
NS = 16
EPAD = 16 * 158 * 128

BM = 512
BM3 = 2000


def _make_edge_scatter(d, k0, chunk, cps):
    mesh = plsc.VectorSubcoreMesh(core_axis_name="c", subcore_axis_name="s")
    rps = NP // NS
    k1 = cps - k0

    @functools.partial(
        pl.kernel,
        mesh=mesh,
        compiler_params=pltpu.CompilerParams(use_tc_tiling_on_sc=False),
        out_type=jax.ShapeDtypeStruct((NC, NP, d), jnp.float32),
        scratch_types=[
            pltpu.VMEM((k0, chunk), jnp.int32),
            pltpu.VMEM((k0, chunk), jnp.int32),
            pltpu.VMEM((chunk, d), jnp.float32),
            pltpu.VMEM_SHARED((NP, d), jnp.float32),
            pltpu.SemaphoreType.DMA,
        ],
    )
    def edge_scatter(tab_hbm, src_hbm, dst_hbm, zeros_hbm, out_hbm,
                     src_v, dst_v, rows_v, acc, gsem):
        cid = lax.axis_index("c")
        sid = lax.axis_index("s")
        pltpu.sync_copy(zeros_hbm.at[pl.ds(sid * rps, rps)],
                        acc.at[pl.ds(sid * rps, rps)])

        @pl.when(cid == 0)
        def _():
            pltpu.sync_copy(src_hbm.at[sid, pl.ds(0, k0)], src_v)
            pltpu.sync_copy(dst_hbm.at[sid, pl.ds(0, k0)], dst_v)

        @pl.when(cid != 0)
        def _():
            pltpu.sync_copy(src_hbm.at[sid, pl.ds(k0, k1)],
                            src_v.at[pl.ds(0, k1)])
            pltpu.sync_copy(dst_hbm.at[sid, pl.ds(k0, k1)],
                            dst_v.at[pl.ds(0, k1)])

        plsc.subcore_barrier()
        nch = jnp.where(cid == 0, k0, k1)

        def body(j, carry):
            pltpu.async_copy(tab_hbm.at[src_v.at[j]], rows_v, gsem).wait()
            pltpu.sync_copy(rows_v, acc.at[dst_v.at[j]], add=True)
            return carry

        lax.fori_loop(0, nch, body, 0)
        plsc.subcore_barrier()
        pltpu.sync_copy(acc.at[pl.ds(sid * rps, rps)],
                        out_hbm.at[cid, pl.ds(sid * rps, rps)])

    return edge_scatter


_scatter1 = _make_edge_scatter(W1TAB, 101, 128, 158)
_scatter2 = _make_edge_scatter(D_OUT, 45, 256, 79)


def _mm1_body(x_ref, wl_ref, wr_ref, ytab_ref, yr_ref):
    i = pl.program_id(0)
    xb = x_ref[...]
    mm = jnp.dot(xb, wl_ref[...], preferred_element_type=jnp.float32)
    rows = i * BM + lax.broadcasted_iota(jnp.int32, (BM, 8), 0)
    cols = lax.broadcasted_iota(jnp.int32, (BM, 8), 1)
    ones = jnp.where((rows < N) & (cols == 0), 1.0, 0.0)
    ytab_ref[...] = jnp.concatenate([mm, ones], axis=1)
    yr_ref[...] = jnp.dot(xb, wr_ref[...], preferred_element_type=jnp.float32)


def _mid_body(a_ref, yr_ref, b1_ref, w2l_ref, w2r_ref, b2_ref,
              y2_ref, r2_ref, dinv_ref):
    i = pl.program_id(0)
    s = a_ref[0] + a_ref[1]
    deg = s[:, 128:129]
    dinv = 1.0 / jnp.maximum(deg, 1.0)
    h = s[:, :128] * dinv + b1_ref[...] + yr_ref[...]
    h = jnp.maximum(h, 0.0)
    rows = i * BM + lax.broadcasted_iota(jnp.int32, (BM, D_HID), 0)
    h = jnp.where(rows < N, h, 0.0)
    y2_ref[...] = jnp.dot(h, w2l_ref[...], preferred_element_type=jnp.float32)
    r2_ref[...] = (jnp.dot(h, w2r_ref[...], preferred_element_type=jnp.float32)
                   + b2_ref[...])
    dinv_ref[...] = jnp.broadcast_to(dinv, (BM, 8))


def _out_body(a_ref, dinv_ref, r2_ref, o_ref):
    s = a_ref[0] + a_ref[1]
    o_ref[...] = s * dinv_ref[...][:, :1] + r2_ref[...]


def kernel(x, edge_index, W1_l, b1_l, W1_r, W2_l, b2_l, W2_r):
    src = edge_index[0].astype(jnp.int32)
    dst = edge_index[1].astype(jnp.int32)
    src_f = jnp.full((EPAD,), N, jnp.int32).at[:E].set(src)
    dst_f = jnp.zeros((EPAD,), jnp.int32).at[:E].set(dst)
    src_p = src_f.reshape(NS, 158, 128)
    dst_p = dst_f.reshape(NS, 158, 128)
    src_p2 = src_f.reshape(NS, 79, 256)
    dst_p2 = dst_f.reshape(NS, 79, 256)
    x_p = jnp.zeros((NP, D_IN), jnp.float32).at[:N].set(x)
    zeros1 = jnp.zeros((NP, W1TAB), jnp.float32)
    zeros2 = jnp.zeros((NP, D_OUT), jnp.float32)
    b1_2d = b1_l.reshape(1, D_HID)
    b2_2d = b2_l.reshape(1, D_OUT)

    ytab, yr1 = pl.pallas_call(
        _mm1_body,
        grid=(NP // BM,),
        in_specs=[
            pl.BlockSpec((BM, D_IN), lambda i: (i, 0)),
            pl.BlockSpec((D_IN, D_HID), lambda i: (0, 0)),
            pl.BlockSpec((D_IN, D_HID), lambda i: (0, 0)),
        ],
        out_specs=[
            pl.BlockSpec((BM, W1TAB), lambda i: (i, 0)),
            pl.BlockSpec((BM, D_HID), lambda i: (i, 0)),
        ],
        out_shape=[
            jax.ShapeDtypeStruct((NP, W1TAB), jnp.float32),
            jax.ShapeDtypeStruct((NP, D_HID), jnp.float32),
        ],
    )(x_p, W1_l, W1_r)

    acc1 = _scatter1(ytab, src_p, dst_p, zeros1)

    y2tab, r2, dinv = pl.pallas_call(
        _mid_body,
        grid=(NP // BM,),
        in_specs=[
            pl.BlockSpec((NC, BM, W1TAB), lambda i: (0, i, 0)),
            pl.BlockSpec((BM, D_HID), lambda i: (i, 0)),
            pl.BlockSpec((1, D_HID), lambda i: (0, 0)),
            pl.BlockSpec((D_HID, D_OUT), lambda i: (0, 0)),
            pl.BlockSpec((D_HID, D_OUT), lambda i: (0, 0)),
            pl.BlockSpec((1, D_OUT), lambda i: (0, 0)),
        ],
        out_specs=[
            pl.BlockSpec((BM, D_OUT), lambda i: (i, 0)),
            pl.BlockSpec((BM, D_OUT), lambda i: (i, 0)),
            pl.BlockSpec((BM, 8), lambda i: (i, 0)),
        ],
        out_shape=[
            jax.ShapeDtypeStruct((NP, D_OUT), jnp.float32),
            jax.ShapeDtypeStruct((NP, D_OUT), jnp.float32),
            jax.ShapeDtypeStruct((NP, 8), jnp.float32),
        ],
    )(acc1, yr1, b1_2d, W2_l, W2_r, b2_2d)

    acc2 = _scatter2(y2tab, src_p2, dst_p2, zeros2)

    out = pl.pallas_call(
        _out_body,
        grid=(N // BM3,),
        in_specs=[
            pl.BlockSpec((NC, BM3, D_OUT), lambda i: (0, i, 0)),
            pl.BlockSpec((BM3, 8), lambda i: (i, 0)),
            pl.BlockSpec((BM3, D_OUT), lambda i: (i, 0)),
        ],
        out_specs=pl.BlockSpec((BM3, D_OUT), lambda i: (i, 0)),
        out_shape=jax.ShapeDtypeStruct((N, D_OUT), jnp.float32),
    )(acc2, dinv, r2)

    return (out, edge_index)


# speedup vs baseline: 1.7219x; 1.0373x over previous
import functools

import jax
import jax.numpy as jnp
from jax import lax
from jax.experimental import pallas as pl
from jax.experimental.pallas import tpu as pltpu
from jax.experimental.pallas import tpu_sc as plsc

N = 10000
NP = 10240
E = 320000
D_IN = 128
D_HID = 128
D_OUT = 40
W1TAB = 136

NC = 2
NS = 16
EPAD = 16 * 158 * 128

BM = 512
BM3 = 2000


def _make_edge_scatter(d, k0, chunk, cps, split=None):
    mesh = plsc.VectorSubcoreMesh(core_axis_name="c", subcore_axis_name="s")
    rps = NP // NS
    k1 = cps - k0
    out_type = (jax.ShapeDtypeStruct((NC, NP, d), jnp.float32) if split is None
                else tuple(jax.ShapeDtypeStruct((NC, NP, w), jnp.float32)
                           for w in split))

    @functools.partial(
        pl.kernel,
        mesh=mesh,
        compiler_params=pltpu.CompilerParams(use_tc_tiling_on_sc=False),
        out_type=out_type,
        scratch_types=[
            pltpu.VMEM((k0, chunk), jnp.int32),
            pltpu.VMEM((k0, chunk), jnp.int32),
            pltpu.VMEM((chunk, d), jnp.float32),
            pltpu.VMEM_SHARED((NP, d), jnp.float32),
            pltpu.SemaphoreType.DMA,
        ],
    )
    def edge_scatter(tab_hbm, src_hbm, dst_hbm, zeros_hbm, *rest):
        if split is None:
            out_refs = rest[:1]
        else:
            out_refs = rest[:2]
        src_v, dst_v, rows_v, acc, gsem = rest[len(out_refs):]
        cid = lax.axis_index("c")
        sid = lax.axis_index("s")
        pltpu.sync_copy(zeros_hbm.at[pl.ds(sid * rps, rps)],
                        acc.at[pl.ds(sid * rps, rps)])

        @pl.when(cid == 0)
        def _():
            pltpu.sync_copy(src_hbm.at[sid, pl.ds(0, k0)], src_v)
            pltpu.sync_copy(dst_hbm.at[sid, pl.ds(0, k0)], dst_v)

        @pl.when(cid != 0)
        def _():
            pltpu.sync_copy(src_hbm.at[sid, pl.ds(k0, k1)],
                            src_v.at[pl.ds(0, k1)])
            pltpu.sync_copy(dst_hbm.at[sid, pl.ds(k0, k1)],
                            dst_v.at[pl.ds(0, k1)])

        plsc.subcore_barrier()
        nch = jnp.where(cid == 0, k0, k1)

        def body(j, carry):
            pltpu.async_copy(tab_hbm.at[src_v.at[j]], rows_v, gsem).wait()
            pltpu.sync_copy(rows_v, acc.at[dst_v.at[j]], add=True)
            return carry

        lax.fori_loop(0, nch, body, 0)
        plsc.subcore_barrier()
        if split is None:
            pltpu.sync_copy(acc.at[pl.ds(sid * rps, rps)],
                            out_refs[0].at[cid, pl.ds(sid * rps, rps)])
        else:
            col = 0
            for w, out_hbm in zip(split, out_refs):
                pltpu.sync_copy(acc.at[pl.ds(sid * rps, rps), pl.ds(col, w)],
                                out_hbm.at[cid, pl.ds(sid * rps, rps)])
                col += w

    return edge_scatter


_scatter1 = _make_edge_scatter(W1TAB, 103, 128, 158, split=(128, 8))
_scatter2 = _make_edge_scatter(D_OUT, 51, 256, 79)


def _mm1_body(x_ref, wl_ref, wr_ref, ytab_ref, yr_ref):
    i = pl.program_id(0)
    xb = x_ref[...]
    mm = jnp.dot(xb, wl_ref[...], preferred_element_type=jnp.float32)
    rows = i * BM + lax.broadcasted_iota(jnp.int32, (BM, 8), 0)
    cols = lax.broadcasted_iota(jnp.int32, (BM, 8), 1)
    ones = jnp.where((rows < N) & (cols == 0), 1.0, 0.0)
    ytab_ref[...] = jnp.concatenate([mm, ones], axis=1)
    yr_ref[...] = jnp.dot(xb, wr_ref[...], preferred_element_type=jnp.float32)


def _mid_body(a_ref, dg_ref, yr_ref, b1_ref, w2l_ref, w2r_ref, b2_ref,
              y2_ref, r2_ref, dinv_ref):
    i = pl.program_id(0)
    s = a_ref[0] + a_ref[1]
    deg = (dg_ref[0] + dg_ref[1])[:, :1]
    dinv = 1.0 / jnp.maximum(deg, 1.0)
    h = s * dinv + b1_ref[...] + yr_ref[...]
    h = jnp.maximum(h, 0.0)
    rows = i * BM + lax.broadcasted_iota(jnp.int32, (BM, D_HID), 0)
    h = jnp.where(rows < N, h, 0.0)
    y2_ref[...] = jnp.dot(h, w2l_ref[...], preferred_element_type=jnp.float32)
    r2_ref[...] = (jnp.dot(h, w2r_ref[...], preferred_element_type=jnp.float32)
                   + b2_ref[...])
    dinv_ref[...] = jnp.broadcast_to(dinv, (BM, 8))


def _out_body(a_ref, dinv_ref, r2_ref, o_ref):
    s = a_ref[0] + a_ref[1]
    o_ref[...] = s * dinv_ref[...][:, :1] + r2_ref[...]


def kernel(x, edge_index, W1_l, b1_l, W1_r, W2_l, b2_l, W2_r):
    src = edge_index[0].astype(jnp.int32)
    dst = edge_index[1].astype(jnp.int32)
    src_f = jnp.full((EPAD,), N, jnp.int32).at[:E].set(src)
    dst_f = jnp.zeros((EPAD,), jnp.int32).at[:E].set(dst)
    src_p = src_f.reshape(NS, 158, 128)
    dst_p = dst_f.reshape(NS, 158, 128)
    src_p2 = src_f.reshape(NS, 79, 256)
    dst_p2 = dst_f.reshape(NS, 79, 256)
    x_p = jnp.zeros((NP, D_IN), jnp.float32).at[:N].set(x)
    zeros1 = jnp.zeros((NP, W1TAB), jnp.float32)
    zeros2 = jnp.zeros((NP, D_OUT), jnp.float32)
    b1_2d = b1_l.reshape(1, D_HID)
    b2_2d = b2_l.reshape(1, D_OUT)

    ytab, yr1 = pl.pallas_call(
        _mm1_body,
        grid=(NP // BM,),
        in_specs=[
            pl.BlockSpec((BM, D_IN), lambda i: (i, 0)),
            pl.BlockSpec((D_IN, D_HID), lambda i: (0, 0)),
            pl.BlockSpec((D_IN, D_HID), lambda i: (0, 0)),
        ],
        out_specs=[
            pl.BlockSpec((BM, W1TAB), lambda i: (i, 0)),
            pl.BlockSpec((BM, D_HID), lambda i: (i, 0)),
        ],
        out_shape=[
            jax.ShapeDtypeStruct((NP, W1TAB), jnp.float32),
            jax.ShapeDtypeStruct((NP, D_HID), jnp.float32),
        ],
    )(x_p, W1_l, W1_r)

    acc1, deg1 = _scatter1(ytab, src_p, dst_p, zeros1)

    y2tab, r2, dinv = pl.pallas_call(
        _mid_body,
        grid=(NP // BM,),
        in_specs=[
            pl.BlockSpec((NC, BM, D_HID), lambda i: (0, i, 0)),
            pl.BlockSpec((NC, BM, 8), lambda i: (0, i, 0)),
            pl.BlockSpec((BM, D_HID), lambda i: (i, 0)),
            pl.BlockSpec((1, D_HID), lambda i: (0, 0)),
            pl.BlockSpec((D_HID, D_OUT), lambda i: (0, 0)),
            pl.BlockSpec((D_HID, D_OUT), lambda i: (0, 0)),
            pl.BlockSpec((1, D_OUT), lambda i: (0, 0)),
        ],
        out_specs=[
            pl.BlockSpec((BM, D_OUT), lambda i: (i, 0)),
            pl.BlockSpec((BM, D_OUT), lambda i: (i, 0)),
            pl.BlockSpec((BM, 8), lambda i: (i, 0)),
        ],
        out_shape=[
            jax.ShapeDtypeStruct((NP, D_OUT), jnp.float32),
            jax.ShapeDtypeStruct((NP, D_OUT), jnp.float32),
            jax.ShapeDtypeStruct((NP, 8), jnp.float32),
        ],
    )(acc1, deg1, yr1, b1_2d, W2_l, W2_r, b2_2d)

    acc2 = _scatter2(y2tab, src_p2, dst_p2, zeros2)

    out = pl.pallas_call(
        _out_body,
        grid=(N // BM3,),
        in_specs=[
            pl.BlockSpec((NC, BM3, D_OUT), lambda i: (0, i, 0)),
            pl.BlockSpec((BM3, 8), lambda i: (i, 0)),
            pl.BlockSpec((BM3, D_OUT), lambda i: (i, 0)),
        ],
        out_specs=pl.BlockSpec((BM3, D_OUT), lambda i: (i, 0)),
        out_shape=jax.ShapeDtypeStruct((N, D_OUT), jnp.float32),
    )(acc2, dinv, r2)

    return (out, edge_index)


# speedup vs baseline: 1.7446x; 1.0132x over previous
import functools

import jax
import jax.numpy as jnp
from jax import lax
from jax.experimental import pallas as pl
from jax.experimental.pallas import tpu as pltpu
from jax.experimental.pallas import tpu_sc as plsc

N = 10000
NP = 10240
E = 320000
D_IN = 128
D_HID = 128
D_OUT = 40
W1TAB = 136

NC = 2
NS = 16
EPAD = 16 * 158 * 128

BM = 512
BM3 = 2000


def _make_edge_scatter(d, k0, chunk, cps, split=None):
    mesh = plsc.VectorSubcoreMesh(core_axis_name="c", subcore_axis_name="s")
    rps = NP // NS
    k1 = cps - k0
    out_type = (jax.ShapeDtypeStruct((NC, NP, d), jnp.float32) if split is None
                else tuple(jax.ShapeDtypeStruct((NC, NP, w), jnp.float32)
                           for w in split))

    @functools.partial(
        pl.kernel,
        mesh=mesh,
        compiler_params=pltpu.CompilerParams(use_tc_tiling_on_sc=False),
        out_type=out_type,
        scratch_types=[
            pltpu.VMEM((k0, chunk), jnp.int32),
            pltpu.VMEM((k0, chunk), jnp.int32),
            pltpu.VMEM((chunk, d), jnp.float32),
            pltpu.VMEM_SHARED((NP, d), jnp.float32),
            pltpu.SemaphoreType.DMA,
        ],
    )
    def edge_scatter(tab_hbm, src_hbm, dst_hbm, *rest):
        if split is None:
            out_refs = rest[:1]
        else:
            out_refs = rest[:2]
        src_v, dst_v, rows_v, acc, gsem = rest[len(out_refs):]
        cid = lax.axis_index("c")
        sid = lax.axis_index("s")

        zcols = list(range(0, d - 16, 16)) + [d - 16]

        def zrow(r, carry):
            for c in zcols:
                rows_v[r, pl.ds(c, 16)] = jnp.zeros((16,), jnp.float32)
            return carry

        lax.fori_loop(0, 128, zrow, 0)
        for t in range(rps // 128):
            pltpu.sync_copy(rows_v.at[pl.ds(0, 128)],
                            acc.at[pl.ds(sid * rps + t * 128, 128)])

        @pl.when(cid == 0)
        def _():
            pltpu.sync_copy(src_hbm.at[sid, pl.ds(0, k0)], src_v)
            pltpu.sync_copy(dst_hbm.at[sid, pl.ds(0, k0)], dst_v)

        @pl.when(cid != 0)
        def _():
            pltpu.sync_copy(src_hbm.at[sid, pl.ds(k0, k1)],
                            src_v.at[pl.ds(0, k1)])
            pltpu.sync_copy(dst_hbm.at[sid, pl.ds(k0, k1)],
                            dst_v.at[pl.ds(0, k1)])

        plsc.subcore_barrier()
        nch = jnp.where(cid == 0, k0, k1)

        def body(j, carry):
            pltpu.async_copy(tab_hbm.at[src_v.at[j]], rows_v, gsem).wait()
            pltpu.sync_copy(rows_v, acc.at[dst_v.at[j]], add=True)
            return carry

        lax.fori_loop(0, nch, body, 0)
        plsc.subcore_barrier()
        if split is None:
            pltpu.sync_copy(acc.at[pl.ds(sid * rps, rps)],
                            out_refs[0].at[cid, pl.ds(sid * rps, rps)])
        else:
            col = 0
            for w, out_hbm in zip(split, out_refs):
                pltpu.sync_copy(acc.at[pl.ds(sid * rps, rps), pl.ds(col, w)],
                                out_hbm.at[cid, pl.ds(sid * rps, rps)])
                col += w

    return edge_scatter


_scatter1 = _make_edge_scatter(W1TAB, 103, 128, 158, split=(128, 8))
_scatter2 = _make_edge_scatter(D_OUT, 51, 256, 79)


def _mm1_body(x_ref, wl_ref, wr_ref, ytab_ref, yr_ref):
    i = pl.program_id(0)
    xb = x_ref[...]
    mm = jnp.dot(xb, wl_ref[...], preferred_element_type=jnp.float32)
    rows = i * BM + lax.broadcasted_iota(jnp.int32, (BM, 8), 0)
    cols = lax.broadcasted_iota(jnp.int32, (BM, 8), 1)
    ones = jnp.where((rows < N) & (cols == 0), 1.0, 0.0)
    ytab_ref[...] = jnp.concatenate([mm, ones], axis=1)
    yr_ref[...] = jnp.dot(xb, wr_ref[...], preferred_element_type=jnp.float32)


def _mid_body(a_ref, dg_ref, yr_ref, b1_ref, w2l_ref, w2r_ref, b2_ref,
              y2_ref, r2_ref, dinv_ref):
    i = pl.program_id(0)
    s = a_ref[0] + a_ref[1]
    deg = (dg_ref[0] + dg_ref[1])[:, :1]
    dinv = 1.0 / jnp.maximum(deg, 1.0)
    h = s * dinv + b1_ref[...] + yr_ref[...]
    h = jnp.maximum(h, 0.0)
    rows = i * BM + lax.broadcasted_iota(jnp.int32, (BM, D_HID), 0)
    h = jnp.where(rows < N, h, 0.0)
    y2_ref[...] = jnp.dot(h, w2l_ref[...], preferred_element_type=jnp.float32)
    r2_ref[...] = (jnp.dot(h, w2r_ref[...], preferred_element_type=jnp.float32)
                   + b2_ref[...])
    dinv_ref[...] = jnp.broadcast_to(dinv, (BM, 8))


def _out_body(a_ref, dinv_ref, r2_ref, o_ref):
    s = a_ref[0] + a_ref[1]
    o_ref[...] = s * dinv_ref[...][:, :1] + r2_ref[...]


def kernel(x, edge_index, W1_l, b1_l, W1_r, W2_l, b2_l, W2_r):
    src = edge_index[0].astype(jnp.int32)
    dst = edge_index[1].astype(jnp.int32)
    src_f = jnp.full((EPAD,), N, jnp.int32).at[:E].set(src)
    dst_f = jnp.zeros((EPAD,), jnp.int32).at[:E].set(dst)
    src_p = src_f.reshape(NS, 158, 128)
    dst_p = dst_f.reshape(NS, 158, 128)
    src_p2 = src_f.reshape(NS, 79, 256)
    dst_p2 = dst_f.reshape(NS, 79, 256)
    x_p = jnp.zeros((NP, D_IN), jnp.float32).at[:N].set(x)
    b1_2d = b1_l.reshape(1, D_HID)
    b2_2d = b2_l.reshape(1, D_OUT)

    ytab, yr1 = pl.pallas_call(
        _mm1_body,
        grid=(NP // BM,),
        in_specs=[
            pl.BlockSpec((BM, D_IN), lambda i: (i, 0)),
            pl.BlockSpec((D_IN, D_HID), lambda i: (0, 0)),
            pl.BlockSpec((D_IN, D_HID), lambda i: (0, 0)),
        ],
        out_specs=[
            pl.BlockSpec((BM, W1TAB), lambda i: (i, 0)),
            pl.BlockSpec((BM, D_HID), lambda i: (i, 0)),
        ],
        out_shape=[
            jax.ShapeDtypeStruct((NP, W1TAB), jnp.float32),
            jax.ShapeDtypeStruct((NP, D_HID), jnp.float32),
        ],
    )(x_p, W1_l, W1_r)

    acc1, deg1 = _scatter1(ytab, src_p, dst_p)

    y2tab, r2, dinv = pl.pallas_call(
        _mid_body,
        grid=(NP // BM,),
        in_specs=[
            pl.BlockSpec((NC, BM, D_HID), lambda i: (0, i, 0)),
            pl.BlockSpec((NC, BM, 8), lambda i: (0, i, 0)),
            pl.BlockSpec((BM, D_HID), lambda i: (i, 0)),
            pl.BlockSpec((1, D_HID), lambda i: (0, 0)),
            pl.BlockSpec((D_HID, D_OUT), lambda i: (0, 0)),
            pl.BlockSpec((D_HID, D_OUT), lambda i: (0, 0)),
            pl.BlockSpec((1, D_OUT), lambda i: (0, 0)),
        ],
        out_specs=[
            pl.BlockSpec((BM, D_OUT), lambda i: (i, 0)),
            pl.BlockSpec((BM, D_OUT), lambda i: (i, 0)),
            pl.BlockSpec((BM, 8), lambda i: (i, 0)),
        ],
        out_shape=[
            jax.ShapeDtypeStruct((NP, D_OUT), jnp.float32),
            jax.ShapeDtypeStruct((NP, D_OUT), jnp.float32),
            jax.ShapeDtypeStruct((NP, 8), jnp.float32),
        ],
    )(acc1, deg1, yr1, b1_2d, W2_l, W2_r, b2_2d)

    acc2 = _scatter2(y2tab, src_p2, dst_p2)

    out = pl.pallas_call(
        _out_body,
        grid=(N // BM3,),
        in_specs=[
            pl.BlockSpec((NC, BM3, D_OUT), lambda i: (0, i, 0)),
            pl.BlockSpec((BM3, 8), lambda i: (i, 0)),
            pl.BlockSpec((BM3, D_OUT), lambda i: (i, 0)),
        ],
        out_specs=pl.BlockSpec((BM3, D_OUT), lambda i: (i, 0)),
        out_shape=jax.ShapeDtypeStruct((N, D_OUT), jnp.float32),
    )(acc2, dinv, r2)

    return (out, edge_index)
